# Initial kernel scaffold; baseline (speedup 1.0000x reference)
import functools

import jax
import jax.numpy as jnp
from jax import lax
from jax.experimental import pallas as pl
from jax.experimental.pallas import tpu as pltpu

HS = 768
NE = 32768
K = 64
B = 32
S = 8
EPS = 1e-6
SCALE = HS ** -0.5
T = 2048


def _q_body(x_ref, wq_ref, wx_ref, q_ref):
    xf = x_ref[...]
    rs = lax.rsqrt(jnp.mean(xf * xf, axis=-1, keepdims=True) + EPS)
    xn = xf * rs * wx_ref[...]
    q_ref[...] = lax.dot_general(xn, wq_ref[...], (((1,), (1,)), ((), ())),
                                 preferred_element_type=jnp.float32)


def _scores_body(q_ref, e_ref, wk_ref, ws_ref, gs_ref):
    ef = e_ref[...]
    rs = lax.rsqrt(jnp.mean(ef * ef, axis=-1, keepdims=True) + EPS)
    en = ef * rs * ws_ref[...]
    kt = lax.dot_general(en, wk_ref[...], (((1,), (1,)), ((), ())),
                         preferred_element_type=jnp.float32)
    st = lax.dot_general(q_ref[...], kt, (((1,), (1,)), ((), ())),
                         preferred_element_type=jnp.float32) * SCALE
    acc = st[0:B, :]
    for s in range(1, S):
        acc = jnp.maximum(acc, st[s * B:(s + 1) * B, :])
    gs_ref[...] = acc


def _compute_gs(X, E, WQ, WK, wx, ws):
    x2 = X.transpose(1, 0, 2).reshape(S * B, HS)
    q = pl.pallas_call(
        _q_body,
        out_shape=jax.ShapeDtypeStruct((S * B, HS), jnp.float32),
    )(x2, WQ, wx.reshape(1, HS))
    grid = (NE // T,)
    gs = pl.pallas_call(
        _scores_body,
        grid=grid,
        in_specs=[
            pl.BlockSpec((S * B, HS), lambda i: (0, 0)),
            pl.BlockSpec((T, HS), lambda i: (i, 0)),
            pl.BlockSpec((HS, HS), lambda i: (0, 0)),
            pl.BlockSpec((1, HS), lambda i: (0, 0)),
        ],
        out_specs=pl.BlockSpec((B, T), lambda i: (0, i)),
        out_shape=jax.ShapeDtypeStruct((B, NE), jnp.float32),
    )(q, E, WK, ws.reshape(1, HS))
    return gs


def kernel(X, E, WQ, WK, wx, ws, wo):
    gs = _compute_gs(X, E, WQ, WK, wx, ws)
    top_k_val, _ = jax.lax.top_k(gs, K)
    cutoff = top_k_val[:, -1:]
    mask = jnp.where(gs >= cutoff, gs, -jnp.inf)
    weights = jax.nn.softmax(mask, axis=-1)
    ef = E.astype(jnp.float32)
    en = ef * lax.rsqrt(jnp.mean(ef * ef, axis=-1, keepdims=True) + EPS) * ws
    O = weights @ en
    O = O * lax.rsqrt(jnp.mean(O * O, axis=-1, keepdims=True) + EPS) * wo
    return (O, weights)


# speedup vs baseline: 3.3426x; 3.3426x over previous
import functools

import jax
import jax.numpy as jnp
from jax import lax
from jax.experimental import pallas as pl
from jax.experimental.pallas import tpu as pltpu

HS = 768
NE = 32768
K = 64
B = 32
S = 8
EPS = 1e-6
SCALE = HS ** -0.5
T = 2048


def _q_body(x_ref, wq_ref, wx_ref, q_ref):
    xf = x_ref[...]
    rs = lax.rsqrt(jnp.mean(xf * xf, axis=-1, keepdims=True) + EPS)
    xn = xf * rs * wx_ref[...]
    q_ref[...] = lax.dot_general(xn, wq_ref[...], (((1,), (1,)), ((), ())),
                                 preferred_element_type=jnp.float32)


def _scores_body(q_ref, e_ref, wk_ref, ws_ref, gs_ref):
    ef = e_ref[...]
    rs = lax.rsqrt(jnp.mean(ef * ef, axis=-1, keepdims=True) + EPS)
    en = ef * rs * ws_ref[...]
    kt = lax.dot_general(en, wk_ref[...], (((1,), (1,)), ((), ())),
                         preferred_element_type=jnp.float32)
    st = lax.dot_general(q_ref[...], kt, (((1,), (1,)), ((), ())),
                         preferred_element_type=jnp.float32) * SCALE
    acc = st[0:B, :]
    for s in range(1, S):
        acc = jnp.maximum(acc, st[s * B:(s + 1) * B, :])
    gs_ref[...] = acc


def _compute_gs(X, E, WQ, WK, wx, ws):
    x2 = X.transpose(1, 0, 2).reshape(S * B, HS)
    q = pl.pallas_call(
        _q_body,
        out_shape=jax.ShapeDtypeStruct((S * B, HS), jnp.float32),
    )(x2, WQ, wx.reshape(1, HS))
    grid = (NE // T,)
    gs = pl.pallas_call(
        _scores_body,
        grid=grid,
        in_specs=[
            pl.BlockSpec((S * B, HS), lambda i: (0, 0)),
            pl.BlockSpec((T, HS), lambda i: (i, 0)),
            pl.BlockSpec((HS, HS), lambda i: (0, 0)),
            pl.BlockSpec((1, HS), lambda i: (0, 0)),
        ],
        out_specs=pl.BlockSpec((B, T), lambda i: (0, i)),
        out_shape=jax.ShapeDtypeStruct((B, NE), jnp.float32),
    )(q, E, WK, ws.reshape(1, HS))
    return gs


NV = NE // 16
CAND = 4096
NCI = CAND // 16
NSEL = 128
HC = HS // 16


def _sc_stage(gs, E, ws, wo):
    from jax.experimental.pallas import tpu_sc as plsc

    mesh = plsc.VectorSubcoreMesh(core_axis_name="c", subcore_axis_name="s")

    def keyify(v):
        k = lax.bitcast_convert_type(v, jnp.int32)
        sm = lax.shift_right_arithmetic(k, 31)
        return k ^ (sm & jnp.int32(0x7FFFFFFF))

    _gd = lax.GatherDimensionNumbers(offset_dims=(), collapsed_slice_dims=(0,),
                                     start_index_map=(0,))

    def take16(v, idx):
        return lax.gather(v, idx[:, None], _gd, slice_sizes=(1,),
                          mode=lax.GatherScatterMode.PROMISE_IN_BOUNDS)

    def newton_rsqrt(x_v):
        kx = lax.bitcast_convert_type(x_v, jnp.int32)
        ky = jnp.int32(0x5F3759DF) - lax.shift_right_logical(kx, 1)
        y = lax.bitcast_convert_type(ky, jnp.float32)
        for _ in range(3):
            y = y * (1.5 - 0.5 * x_v * y * y)
        return y

    @functools.partial(
        pl.kernel,
        out_type=[jax.ShapeDtypeStruct((B, NE), jnp.float32),
                  jax.ShapeDtypeStruct((B, HS), jnp.float32)],
        mesh=mesh,
        compiler_params=pltpu.CompilerParams(needs_layout_passes=False),
        scratch_types=[
            pltpu.VMEM((NE,), jnp.float32),
            pltpu.VMEM((16 * 256,), jnp.int32),
            pltpu.VMEM((CAND,), jnp.float32),
            pltpu.VMEM((CAND,), jnp.int32),
            pltpu.VMEM((NSEL,), jnp.float32),
            pltpu.VMEM((NSEL,), jnp.int32),
            pltpu.VMEM((64, HS), jnp.float32),
            pltpu.VMEM((HS,), jnp.float32),
            pltpu.VMEM((HS,), jnp.float32),
            pltpu.VMEM((HS,), jnp.float32),
            pltpu.SemaphoreType.DMA,
        ],
    )
    def sck(gs_hbm, e_hbm, ws_hbm, wo_hbm, w_out, o_out,
            row_buf, hist, cand_v, cand_i, sel_w, sel_i, erows, oacc, wsv,
            wov, sem):
        zi = jnp.zeros((16,), jnp.int32)
        zf = jnp.zeros((16,), jnp.float32)
        onesi = jnp.ones((16,), jnp.int32)
        row = lax.axis_index("s") * 2 + lax.axis_index("c")
        pltpu.sync_copy(gs_hbm.at[row], row_buf)
        pltpu.sync_copy(ws_hbm, wsv)
        pltpu.sync_copy(wo_hbm, wov)

        lanes = lax.broadcasted_iota(jnp.int32, (16,), 0)
        laddr = lanes * 256

        def clear_hist():
            def bd(t, c):
                hist[pl.ds(t * 16, 16)] = zi
                return c
            lax.fori_loop(0, 256, bd, 0)

        def scan_hist(r_v):
            fifteen = jnp.full((16,), 15, jnp.int32)

            def bd(gi, c):
                found_v, bsel_v, cgt_v, run_v = c
                g = 15 - gi
                tv = zi
                for lane in range(16):
                    tv = tv + hist[pl.ds(lane * 256 + g * 16, 16)]
                rv = lax.rev(tv, (0,))
                cs = plsc.cumsum(rv)
                tot = run_v + cs
                m = tot >= r_v
                npop = plsc.all_reduce_population_count(m)
                j_v = plsc.all_reduce_ffs(m)
                anyv = jnp.where(npop > 0, 1, 0).astype(jnp.int32) * onesi
                upd = anyv * (1 - found_v)
                jc = jnp.clip(j_v, 0, 15)
                rv_at_j = take16(rv, jc)
                tot_at_j = take16(tot, jc)
                grp_tot = take16(cs, fifteen)
                bsel_v = bsel_v + upd * (g * 16 + fifteen - jc)
                cgt_v = cgt_v + upd * (tot_at_j - rv_at_j)
                run_v = run_v + (1 - found_v) * grp_tot
                found_v = jnp.maximum(found_v, anyv)
                return (found_v, bsel_v, cgt_v, run_v)

            out = lax.fori_loop(0, 16, bd, (zi, zi, zi, zi))
            return out[1], out[2]

        clear_hist()

        def p1_body(t, macc):
            v = row_buf[pl.ds(t * 16, 16)]
            kk = keyify(v)
            b = lax.shift_right_arithmetic(kk, 24) + 128
            plsc.addupdate_scatter(hist, [laddr + b], onesi)
            return jnp.maximum(macc, v)

        macc = lax.fori_loop(0, NV, p1_body,
                             jnp.full((16,), -jnp.inf, jnp.float32))
        m_max = jnp.max(macc)

        rk = jnp.full((16,), K, jnp.int32)
        b1v, cgt1 = scan_hist(rk)
        p1v = b1v - 128
        r2 = rk - cgt1

        clear_hist()

        def p2_body(t, c):
            v = row_buf[pl.ds(t * 16, 16)]
            kk = keyify(v)
            m = lax.shift_right_arithmetic(kk, 24) == p1v
            b = lax.shift_right_arithmetic(kk, 16) & 0xFF
            plsc.addupdate_scatter(hist, [laddr + b], onesi, mask=m)
            return c

        lax.fori_loop(0, NV, p2_body, 0)
        b2v, cgt2 = scan_hist(r2)
        prefix2 = p1v * 256 + b2v
        r3 = r2 - cgt2

        thr2 = prefix2 * 65536

        def pc_body(t, off_v):
            v = row_buf[pl.ds(t * 16, 16)]
            kk = keyify(v)
            m = kk >= thr2
            mi = m.astype(jnp.int32)
            cs = plsc.cumsum(mi)
            pos = off_v + cs - mi
            mm = m & (pos < CAND)
            plsc.store_scatter(cand_v, [pos], v, mask=mm)
            gidx = t * 16 + lanes
            plsc.store_scatter(cand_i, [pos], gidx, mask=mm)
            return off_v + plsc.all_reduce_population_count(m)

        ncand_v = lax.fori_loop(0, NV, pc_body, zi)

        def cand_hist(shift_hi, shift_lo, prefix_v):
            def bd(t, c):
                v = cand_v[pl.ds(t * 16, 16)]
                kk = keyify(v)
                posv = t * 16 + lanes
                pm = posv < ncand_v
                m = pm & (lax.shift_right_arithmetic(kk, shift_hi) == prefix_v)
                b = lax.shift_right_arithmetic(kk, shift_lo) & 0xFF
                plsc.addupdate_scatter(hist, [laddr + b], onesi, mask=m)
                return c
            lax.fori_loop(0, NCI, bd, 0)

        clear_hist()
        cand_hist(16, 8, prefix2)
        b3v, cgt3 = scan_hist(r3)
        prefix3 = prefix2 * 256 + b3v
        r4 = r3 - cgt3

        clear_hist()
        cand_hist(8, 0, prefix3)
        b4v, _ = scan_hist(r4)
        cutoff_v = prefix3 * 256 + b4v

        for g in range(8):
            sel_i[g * 16:(g + 1) * 16] = zi

        def ps_body(t, off_v):
            v = cand_v[pl.ds(t * 16, 16)]
            kk = keyify(v)
            posv = t * 16 + lanes
            pm = posv < ncand_v
            m = pm & (kk >= cutoff_v)
            mi = m.astype(jnp.int32)
            cs = plsc.cumsum(mi)
            pos = off_v + cs - mi
            mm = m & (pos < NSEL)
            plsc.store_scatter(sel_w, [pos], v, mask=mm)
            ii = cand_i[pl.ds(t * 16, 16)]
            plsc.store_scatter(sel_i, [pos], ii, mask=mm)
            return off_v + plsc.all_reduce_population_count(m)

        n_v = lax.fori_loop(0, NCI, ps_body, zi)
        n_s = jnp.max(n_v)

        zacc = zf
        evs = []
        for g in range(8):
            vm = (g * 16 + lanes) < n_v
            v = sel_w[g * 16:(g + 1) * 16]
            e = jnp.where(vm, jnp.exp(v - m_max), 0.0)
            evs.append(e)
            zacc = zacc + e
        zsum = jnp.sum(zacc)
        for g in range(8):
            sel_w[g * 16:(g + 1) * 16] = evs[g] / zsum

        def mz_body(t, c):
            row_buf[pl.ds(t * 16, 16)] = zf
            return c

        lax.fori_loop(0, NV, mz_body, 0)
        for g in range(8):
            vm = (g * 16 + lanes) < n_v
            w = sel_w[g * 16:(g + 1) * 16]
            ii = sel_i[g * 16:(g + 1) * 16]
            plsc.store_scatter(row_buf, [ii], w, mask=vm)
        pltpu.sync_copy(row_buf, w_out.at[row])

        for cc in range(HC):
            oacc[cc * 16:(cc + 1) * 16] = zf

        def obatch(base):
            pltpu.async_copy(e_hbm.at[sel_i.at[pl.ds(base, 64)]], erows,
                             sem).wait()

            def row_body(i, c):
                sq = zf
                for cc in range(HC):
                    v = erows[i, pl.ds(cc * 16, 16)]
                    sq = sq + v * v
                msq = jnp.sum(sq) * (1.0 / HS) + EPS
                y = newton_rsqrt(msq * jnp.ones((16,), jnp.float32))
                off = base + i
                wv = sel_w[pl.ds(off & (-16), 16)]
                coef_v = y * take16(wv, (off & 15) * onesi)
                for cc in range(HC):
                    oacc[cc * 16:(cc + 1) * 16] = (
                        oacc[cc * 16:(cc + 1) * 16]
                        + coef_v * erows[i, pl.ds(cc * 16, 16)])
                return c

            lax.fori_loop(0, 64, row_body, 0)

        obatch(0)

        @pl.when(n_s > 64)
        def _():
            obatch(64)

        sq = zf
        for cc in range(HC):
            v = oacc[cc * 16:(cc + 1) * 16] * wsv[cc * 16:(cc + 1) * 16]
            oacc[cc * 16:(cc + 1) * 16] = v
            sq = sq + v * v
        msq = jnp.sum(sq) * (1.0 / HS) + EPS
        y = newton_rsqrt(msq * jnp.ones((16,), jnp.float32))
        for cc in range(HC):
            oacc[cc * 16:(cc + 1) * 16] = (
                oacc[cc * 16:(cc + 1) * 16] * y * wov[cc * 16:(cc + 1) * 16])
        pltpu.sync_copy(oacc, o_out.at[row])

    return sck(gs, E, ws, wo)


def kernel(X, E, WQ, WK, wx, ws, wo):
    gs = _compute_gs(X, E, WQ, WK, wx, ws)
    weights, O = _sc_stage(gs, E, ws, wo)
    return (O, weights)


# speedup vs baseline: 3.4394x; 1.0290x over previous
import functools

import jax
import jax.numpy as jnp
from jax import lax
from jax.experimental import pallas as pl
from jax.experimental.pallas import tpu as pltpu

HS = 768
NE = 32768
K = 64
B = 32
S = 8
EPS = 1e-6
SCALE = HS ** -0.5
T = 2048


def _q_body(x_ref, wq_ref, wx_ref, q_ref):
    xf = x_ref[...]
    rs = lax.rsqrt(jnp.mean(xf * xf, axis=-1, keepdims=True) + EPS)
    xn = xf * rs * wx_ref[...]
    q_ref[...] = lax.dot_general(xn, wq_ref[...], (((1,), (1,)), ((), ())),
                                 preferred_element_type=jnp.float32)


def _scores_body(q_ref, e_ref, wk_ref, ws_ref, gs_ref):
    ef = e_ref[...]
    rs = lax.rsqrt(jnp.mean(ef * ef, axis=-1, keepdims=True) + EPS)
    en = ef * rs * ws_ref[...]
    kt = lax.dot_general(en, wk_ref[...], (((1,), (1,)), ((), ())),
                         preferred_element_type=jnp.float32)
    st = lax.dot_general(q_ref[...], kt, (((1,), (1,)), ((), ())),
                         preferred_element_type=jnp.float32) * SCALE
    acc = st[0:B, :]
    for s in range(1, S):
        acc = jnp.maximum(acc, st[s * B:(s + 1) * B, :])
    gs_ref[...] = acc


def _compute_gs(X, E, WQ, WK, wx, ws):
    x2 = X.transpose(1, 0, 2).reshape(S * B, HS)
    q = pl.pallas_call(
        _q_body,
        out_shape=jax.ShapeDtypeStruct((S * B, HS), jnp.float32),
    )(x2, WQ, wx.reshape(1, HS))
    grid = (NE // T,)
    gs = pl.pallas_call(
        _scores_body,
        grid=grid,
        in_specs=[
            pl.BlockSpec((S * B, HS), lambda i: (0, 0)),
            pl.BlockSpec((T, HS), lambda i: (i, 0)),
            pl.BlockSpec((HS, HS), lambda i: (0, 0)),
            pl.BlockSpec((1, HS), lambda i: (0, 0)),
        ],
        out_specs=pl.BlockSpec((B, T), lambda i: (0, i)),
        out_shape=jax.ShapeDtypeStruct((B, NE), jnp.float32),
    )(q, E, WK, ws.reshape(1, HS))
    return gs


NV = NE // 16
CAND = 4096
NCI = CAND // 16
NSEL = 128
HC = HS // 16


def _sc_stage(gs, E, ws, wo):
    from jax.experimental.pallas import tpu_sc as plsc

    mesh = plsc.VectorSubcoreMesh(core_axis_name="c", subcore_axis_name="s")

    def keyify(v):
        k = lax.bitcast_convert_type(v, jnp.int32)
        sm = lax.shift_right_arithmetic(k, 31)
        return k ^ (sm & jnp.int32(0x7FFFFFFF))

    _gd = lax.GatherDimensionNumbers(offset_dims=(), collapsed_slice_dims=(0,),
                                     start_index_map=(0,))

    def take16(v, idx):
        return lax.gather(v, idx[:, None], _gd, slice_sizes=(1,),
                          mode=lax.GatherScatterMode.PROMISE_IN_BOUNDS)

    def newton_rcp(x_v):
        kx = lax.bitcast_convert_type(x_v, jnp.int32)
        ky = jnp.int32(0x7EF477D5) - kx
        y = lax.bitcast_convert_type(ky, jnp.float32)
        for _ in range(3):
            y = y * (2.0 - x_v * y)
        return y

    _L2E = 1.4426950408889634
    _EC0 = 0.6931471805599453
    _EC1 = 1.9082149292705877e-10

    def exp_hp(x_v):
        ni0 = (x_v * _L2E - 0.5).astype(jnp.int32)
        n = ni0.astype(jnp.float32)
        f = (x_v - n * _EC0) - n * _EC1
        p = 1.9841269841269841e-4
        p = p * f + 1.3888888888888889e-3
        p = p * f + 8.3333333333333332e-3
        p = p * f + 4.1666666666666664e-2
        p = p * f + 1.6666666666666666e-1
        p = p * f + 5.0e-1
        p = p * f + 1.0
        p = p * f + 1.0
        ni = jnp.maximum(ni0, -126)
        sc = lax.bitcast_convert_type((ni + 127) * 8388608, jnp.float32)
        return p * sc

    def newton_rsqrt(x_v):
        kx = lax.bitcast_convert_type(x_v, jnp.int32)
        ky = jnp.int32(0x5F3759DF) - lax.shift_right_logical(kx, 1)
        y = lax.bitcast_convert_type(ky, jnp.float32)
        for _ in range(3):
            y = y * (1.5 - 0.5 * x_v * y * y)
        return y

    @functools.partial(
        pl.kernel,
        out_type=[jax.ShapeDtypeStruct((B, NE), jnp.float32),
                  jax.ShapeDtypeStruct((B, HS), jnp.float32)],
        mesh=mesh,
        compiler_params=pltpu.CompilerParams(needs_layout_passes=False),
        scratch_types=[
            pltpu.VMEM((NE,), jnp.float32),
            pltpu.VMEM((16 * 256,), jnp.int32),
            pltpu.VMEM((CAND,), jnp.float32),
            pltpu.VMEM((CAND,), jnp.int32),
            pltpu.VMEM((NSEL,), jnp.float32),
            pltpu.VMEM((NSEL,), jnp.int32),
            pltpu.VMEM((64, HS), jnp.float32),
            pltpu.VMEM((HS,), jnp.float32),
            pltpu.VMEM((HS,), jnp.float32),
            pltpu.VMEM((HS,), jnp.float32),
            pltpu.SemaphoreType.DMA,
        ],
    )
    def sck(gs_hbm, e_hbm, ws_hbm, wo_hbm, w_out, o_out,
            row_buf, hist, cand_v, cand_i, sel_w, sel_i, erows, oacc, wsv,
            wov, sem):
        zi = jnp.zeros((16,), jnp.int32)
        zf = jnp.zeros((16,), jnp.float32)
        onesi = jnp.ones((16,), jnp.int32)
        row = lax.axis_index("s") * 2 + lax.axis_index("c")
        pltpu.sync_copy(gs_hbm.at[row], row_buf)
        pltpu.sync_copy(ws_hbm, wsv)
        pltpu.sync_copy(wo_hbm, wov)

        lanes = lax.broadcasted_iota(jnp.int32, (16,), 0)
        laddr = lanes * 256

        def clear_hist():
            def bd(t, c):
                hist[pl.ds(t * 16, 16)] = zi
                return c
            lax.fori_loop(0, 256, bd, 0)

        def scan_hist(r_v):
            fifteen = jnp.full((16,), 15, jnp.int32)

            def bd(gi, c):
                found_v, bsel_v, cgt_v, run_v = c
                g = 15 - gi
                tv = zi
                for lane in range(16):
                    tv = tv + hist[pl.ds(lane * 256 + g * 16, 16)]
                rv = lax.rev(tv, (0,))
                cs = plsc.cumsum(rv)
                tot = run_v + cs
                m = tot >= r_v
                npop = plsc.all_reduce_population_count(m)
                j_v = plsc.all_reduce_ffs(m)
                anyv = jnp.where(npop > 0, 1, 0).astype(jnp.int32) * onesi
                upd = anyv * (1 - found_v)
                jc = jnp.clip(j_v, 0, 15)
                rv_at_j = take16(rv, jc)
                tot_at_j = take16(tot, jc)
                grp_tot = take16(cs, fifteen)
                bsel_v = bsel_v + upd * (g * 16 + fifteen - jc)
                cgt_v = cgt_v + upd * (tot_at_j - rv_at_j)
                run_v = run_v + (1 - found_v) * grp_tot
                found_v = jnp.maximum(found_v, anyv)
                return (found_v, bsel_v, cgt_v, run_v)

            out = lax.fori_loop(0, 16, bd, (zi, zi, zi, zi))
            return out[1], out[2]

        clear_hist()

        def p1_body(t, macc):
            for u in range(2):
                v = row_buf[pl.ds(t * 32 + u * 16, 16)]
                kk = keyify(v)
                b = lax.shift_right_arithmetic(kk, 24) + 128
                plsc.addupdate_scatter(hist, [laddr + b], onesi)
                macc = jnp.maximum(macc, v)
            return macc

        macc = lax.fori_loop(0, NV // 2, p1_body,
                             jnp.full((16,), -jnp.inf, jnp.float32))
        m_max = jnp.max(macc)

        rk = jnp.full((16,), K, jnp.int32)
        b1v, cgt1 = scan_hist(rk)
        p1v = b1v - 128
        r2 = rk - cgt1

        clear_hist()

        def p2_body(t, c):
            for u in range(2):
                v = row_buf[pl.ds(t * 32 + u * 16, 16)]
                kk = keyify(v)
                m = lax.shift_right_arithmetic(kk, 24) == p1v
                b = lax.shift_right_arithmetic(kk, 16) & 0xFF
                plsc.addupdate_scatter(hist, [laddr + b], onesi, mask=m)
            return c

        lax.fori_loop(0, NV // 2, p2_body, 0)
        b2v, cgt2 = scan_hist(r2)
        prefix2 = p1v * 256 + b2v
        r3 = r2 - cgt2

        thr2 = prefix2 * 65536

        def pc_body(t, off_v):
            for u in range(2):
                v = row_buf[pl.ds(t * 32 + u * 16, 16)]
                kk = keyify(v)
                m = kk >= thr2
                mi = m.astype(jnp.int32)
                cs = plsc.cumsum(mi)
                pos = off_v + cs - mi
                mm = m & (pos < CAND)
                plsc.store_scatter(cand_v, [pos], v, mask=mm)
                gidx = t * 32 + u * 16 + lanes
                plsc.store_scatter(cand_i, [pos], gidx, mask=mm)
                row_buf[pl.ds(t * 32 + u * 16, 16)] = zf
                off_v = off_v + plsc.all_reduce_population_count(m)
            return off_v

        ncand_v = lax.fori_loop(0, NV // 2, pc_body, zi)

        def cand_hist(shift_hi, shift_lo, prefix_v):
            def bd(t, c):
                v = cand_v[pl.ds(t * 16, 16)]
                kk = keyify(v)
                posv = t * 16 + lanes
                pm = posv < ncand_v
                m = pm & (lax.shift_right_arithmetic(kk, shift_hi) == prefix_v)
                b = lax.shift_right_arithmetic(kk, shift_lo) & 0xFF
                plsc.addupdate_scatter(hist, [laddr + b], onesi, mask=m)
                return c
            lax.fori_loop(0, NCI, bd, 0)

        clear_hist()
        cand_hist(16, 8, prefix2)
        b3v, cgt3 = scan_hist(r3)
        prefix3 = prefix2 * 256 + b3v
        r4 = r3 - cgt3

        clear_hist()
        cand_hist(8, 0, prefix3)
        b4v, _ = scan_hist(r4)
        cutoff_v = prefix3 * 256 + b4v

        for g in range(8):
            sel_i[g * 16:(g + 1) * 16] = zi

        def ps_body(t, off_v):
            v = cand_v[pl.ds(t * 16, 16)]
            kk = keyify(v)
            posv = t * 16 + lanes
            pm = posv < ncand_v
            m = pm & (kk >= cutoff_v)
            mi = m.astype(jnp.int32)
            cs = plsc.cumsum(mi)
            pos = off_v + cs - mi
            mm = m & (pos < NSEL)
            plsc.store_scatter(sel_w, [pos], v, mask=mm)
            ii = cand_i[pl.ds(t * 16, 16)]
            plsc.store_scatter(sel_i, [pos], ii, mask=mm)
            return off_v + plsc.all_reduce_population_count(m)

        n_v = lax.fori_loop(0, NCI, ps_body, zi)
        n_s = jnp.max(n_v)

        zacc = zf
        evs = []
        for g in range(8):
            vm = (g * 16 + lanes) < n_v
            v = sel_w[g * 16:(g + 1) * 16]
            e = jnp.where(vm, exp_hp(v - m_max), 0.0)
            evs.append(e)
            zacc = zacc + e
        zsum = jnp.sum(zacc)
        zinv = newton_rcp(zsum * jnp.ones((16,), jnp.float32))
        for g in range(8):
            sel_w[g * 16:(g + 1) * 16] = evs[g] * zinv

        for g in range(8):
            vm = (g * 16 + lanes) < n_v
            w = sel_w[g * 16:(g + 1) * 16]
            ii = sel_i[g * 16:(g + 1) * 16]
            plsc.store_scatter(row_buf, [ii], w, mask=vm)
        pltpu.sync_copy(row_buf, w_out.at[row])

        for cc in range(HC):
            oacc[cc * 16:(cc + 1) * 16] = zf

        def obatch(base):
            pltpu.async_copy(e_hbm.at[sel_i.at[pl.ds(base, 64)]], erows,
                             sem).wait()

            def row_body(i, c):
                sq = zf
                for cc in range(HC):
                    v = erows[i, pl.ds(cc * 16, 16)]
                    sq = sq + v * v
                msq = jnp.sum(sq) * (1.0 / HS) + EPS
                y = newton_rsqrt(msq * jnp.ones((16,), jnp.float32))
                off = base + i
                wv = sel_w[pl.ds(off & (-16), 16)]
                coef_v = y * take16(wv, (off & 15) * onesi)
                for cc in range(HC):
                    oacc[cc * 16:(cc + 1) * 16] = (
                        oacc[cc * 16:(cc + 1) * 16]
                        + coef_v * erows[i, pl.ds(cc * 16, 16)])
                return c

            lax.fori_loop(0, 64, row_body, 0)

        obatch(0)

        @pl.when(n_s > 64)
        def _():
            obatch(64)

        sq = zf
        for cc in range(HC):
            v = oacc[cc * 16:(cc + 1) * 16] * wsv[cc * 16:(cc + 1) * 16]
            oacc[cc * 16:(cc + 1) * 16] = v
            sq = sq + v * v
        msq = jnp.sum(sq) * (1.0 / HS) + EPS
        y = newton_rsqrt(msq * jnp.ones((16,), jnp.float32))
        for cc in range(HC):
            oacc[cc * 16:(cc + 1) * 16] = (
                oacc[cc * 16:(cc + 1) * 16] * y * wov[cc * 16:(cc + 1) * 16])
        pltpu.sync_copy(oacc, o_out.at[row])

    return sck(gs, E, ws, wo)


def kernel(X, E, WQ, WK, wx, ws, wo):
    gs = _compute_gs(X, E, WQ, WK, wx, ws)
    weights, O = _sc_stage(gs, E, ws, wo)
    return (O, weights)
